# baseline (speedup 1.0000x reference)
import functools

import jax
import jax.numpy as jnp
from jax import lax
from jax.experimental import pallas as pl
from jax.experimental.pallas import tpu as pltpu
from jax.experimental.pallas import tpu_sc as plsc

B, N, G, K = 16, 8192, 128, 32
L = 16
GH = G // 2
NV = N // L
CHUNK = 16
NCHUNK = NV // CHUNK
INF = 3.0e38
SLOT = 16384


def _fps_body(xt_ref, centers_ref, dist_ref):
    X = xt_ref[0]
    Y = xt_ref[1]
    Z = xt_ref[2]
    lane = jax.lax.broadcasted_iota(jnp.int32, (B, N), 1)
    dist_ref[...] = jnp.full((B, N), 1e10, dtype=jnp.float32)

    def step(i, far):
        mask = lane == far
        cx = jnp.sum(jnp.where(mask, X, 0.0), axis=1, keepdims=True)
        cy = jnp.sum(jnp.where(mask, Y, 0.0), axis=1, keepdims=True)
        cz = jnp.sum(jnp.where(mask, Z, 0.0), axis=1, keepdims=True)
        centers_ref[i, :, :] = jnp.concatenate([cx, cy, cz], axis=1)
        dx = X - cx
        dy = Y - cy
        dz = Z - cz
        d = dx * dx + dy * dy + dz * dz
        nd = jnp.minimum(dist_ref[...], d)
        dist_ref[...] = nd
        m = jnp.max(nd, axis=1, keepdims=True)
        far2 = jnp.min(jnp.where(nd == m, lane, N), axis=1, keepdims=True)
        return far2

    jax.lax.fori_loop(0, G, step, jnp.zeros((B, 1), jnp.int32))


def _fps_centers(xyz):
    xt = jnp.transpose(xyz, (2, 0, 1))
    centers_gb3 = pl.pallas_call(
        _fps_body,
        out_shape=jax.ShapeDtypeStruct((G, B, 3), jnp.float32),
        scratch_shapes=[pltpu.VMEM((B, N), jnp.float32)],
    )(xt)
    return centers_gb3


def _knn_body(xt_hbm, ct_hbm, out_hbm, xv, yv, zv, cv, db, cd, cix,
              cd2, cix2, cntb, exb, taub, pbuf):
    c_ax = lax.axis_index("c")
    s_ax = lax.axis_index("s")
    b = s_ax
    gh = c_ax

    pltpu.sync_copy(xt_hbm.at[pl.ds(b * (3 * N), N)], xv)
    pltpu.sync_copy(xt_hbm.at[pl.ds(b * (3 * N) + N, N)], yv)
    pltpu.sync_copy(xt_hbm.at[pl.ds(b * (3 * N) + 2 * N, N)], zv)
    pltpu.sync_copy(ct_hbm.at[pl.ds(b * (3 * G), 3 * G)], cv)

    lane = lax.broadcasted_iota(jnp.int32, (L,), 0)
    inf_v = jnp.full((L,), INF, dtype=jnp.float32)
    bigi_v = jnp.full((L,), 1 << 30, dtype=jnp.int32)
    zero_i = jnp.zeros((L,), jnp.int32)

    def per_group(g, _):
        gg = gh * GH + g
        ggv = jnp.full((L,), 0, jnp.int32) + gg
        cgx = plsc.load_gather(cv, [ggv])
        cgy = plsc.load_gather(cv, [ggv + G])
        cgz = plsc.load_gather(cv, [ggv + 2 * G])

        def dist_at(base, v):
            sl = pl.ds(base + v * L, L)
            dx = xv[sl] - cgx
            dy = yv[sl] - cgy
            dz = zv[sl] - cgz
            return dx * dx + dy * dy + dz * dz

        m1a = m2a = m1b = m2b = inf_v
        for v in range(CHUNK):
            d = dist_at(0, v)
            if v % 2 == 0:
                m2a = jnp.minimum(m2a, jnp.maximum(m1a, d))
                m1a = jnp.minimum(m1a, d)
            else:
                m2b = jnp.minimum(m2b, jnp.maximum(m1b, d))
                m1b = jnp.minimum(m1b, d)
        tau0 = jnp.max(jnp.minimum(m2a, m2b))

        def sweep(c, carry):
            m1a, m2a, m1b, m2b, tau = carry
            base = c * (CHUNK * L)
            plsc.store_scatter(taub, [zero_i + c],
                               jnp.zeros((L,), jnp.float32) + tau,
                               mask=lane == 0)
            cnt_vec = zero_i
            for v in range(CHUNK):
                d = dist_at(base, v)
                db[pl.ds(base + v * L, L)] = d
                msk = d <= tau
                pc = plsc.all_reduce_population_count(msk)
                cnt_vec = jnp.where(lane == v, pc, cnt_vec)
                if v % 2 == 0:
                    m2a = jnp.minimum(m2a, jnp.maximum(m1a, d))
                    m1a = jnp.minimum(m1a, d)
                else:
                    m2b = jnp.minimum(m2b, jnp.maximum(m1b, d))
                    m1b = jnp.minimum(m1b, d)
            cntb[pl.ds(c * L, L)] = cnt_vec
            tau = jnp.max(jnp.minimum(m2a, m2b))
            return m1a, m2a, m1b, m2b, tau

        m1a, m2a, m1b, m2b, _ = lax.fori_loop(
            0, NCHUNK, sweep, (inf_v, inf_v, inf_v, inf_v, tau0))
        tau_f = jnp.max(jnp.minimum(m2a, m2b))

        def prefix(c, carry_tot):
            cnt_vec = cntb[pl.ds(c * L, L)]
            cs = plsc.cumsum(cnt_vec)
            exb[pl.ds(c * L, L)] = carry_tot + (cs - cnt_vec)
            return carry_tot + cs[L - 1]

        tot_v = lax.fori_loop(0, NCHUNK, prefix, zero_i)
        crun = jnp.max(tot_v)

        def scat(c, _):
            base = c * (CHUNK * L)
            tau_c = plsc.load_gather(taub, [zero_i + c])
            exv = exb[pl.ds(c * L, L)]
            for v in range(CHUNK):
                sl = pl.ds(base + v * L, L)
                d = db[sl]
                msk = d <= tau_c
                mi32 = msk.astype(jnp.int32)
                pos = exv[v] + plsc.cumsum(mi32) - 1
                plsc.store_scatter(cd, [pos], d, mask=msk)
                plsc.store_scatter(cix, [pos], lane + (base + v * L), mask=msk)
            return 0

        lax.fori_loop(0, NCHUNK, scat, 0)

        def mini(v, cur2):
            sl = pl.ds(v * L, L)
            d = cd[sl]
            iv = cix[sl]
            msk = (d <= tau_f) & ((v * L + lane) < crun)
            plsc.store_compressed(cd2.at[pl.ds(cur2, L)], d, mask=msk)
            plsc.store_compressed(cix2.at[pl.ds(cur2, L)], iv, mask=msk)
            return cur2 + plsc.all_reduce_population_count(msk)[0]

        cursor = lax.fori_loop(0, (crun + (L - 1)) // L, mini, jnp.int32(0))
        plsc.store_scatter(cd2, [cursor + lane], inf_v)

        nvc = (cursor + (L - 1)) // L

        for t in range(K // L):
            accv = jnp.zeros((L,), jnp.int32)
            for jj in range(L):
                def scan(v, carry):
                    m, mc = carry
                    sl = pl.ds(v * L, L)
                    dv = cd2[sl]
                    cmb = cix2[sl] * SLOT + (lane + v * L)
                    upd = (dv < m) | ((dv == m) & (cmb < mc))
                    m = jnp.where(upd, dv, m)
                    mc = jnp.where(upd, cmb, mc)
                    return m, mc

                m, mc = lax.fori_loop(0, nvc, scan, (inf_v, bigi_v))
                dmin = jnp.min(m)
                cmbmin = jnp.min(jnp.where(m == dmin, mc, bigi_v))
                imin = cmbmin // SLOT
                pmin = cmbmin - imin * SLOT
                plsc.store_scatter(cd2, [zero_i + pmin], inf_v, mask=lane == 0)
                accv = jnp.where(lane == jj, imin, accv)

            px = plsc.load_gather(xv, [accv]) - cgx
            py = plsc.load_gather(yv, [accv]) - cgy
            pz = plsc.load_gather(zv, [accv]) - cgz
            pos = (g * K + t * L) * 3 + lane * 3
            plsc.store_scatter(pbuf, [pos], px)
            plsc.store_scatter(pbuf, [pos + 1], py)
            plsc.store_scatter(pbuf, [pos + 2], pz)
        return 0

    lax.fori_loop(0, GH, per_group, 0)
    pltpu.sync_copy(pbuf, out_hbm.at[pl.ds((b * 2 + gh) * (GH * K * 3), GH * K * 3)])


def _knn_patch_sc(xyz, centers_gb3):
    xt = jnp.transpose(xyz, (0, 2, 1)).reshape(B * 3 * N)
    ct = jnp.transpose(centers_gb3, (1, 2, 0)).reshape(B * 3 * G)
    mesh = plsc.VectorSubcoreMesh(core_axis_name="c", subcore_axis_name="s")
    out = pl.kernel(
        _knn_body,
        out_type=jax.ShapeDtypeStruct((B * 2 * GH * K * 3,), jnp.float32),
        mesh=mesh,
        compiler_params=pltpu.CompilerParams(needs_layout_passes=False),
        scratch_types=[
            pltpu.VMEM((N,), jnp.float32),
            pltpu.VMEM((N,), jnp.float32),
            pltpu.VMEM((N,), jnp.float32),
            pltpu.VMEM((3 * G,), jnp.float32),
            pltpu.VMEM((N,), jnp.float32),
            pltpu.VMEM((N + L,), jnp.float32),
            pltpu.VMEM((N + L,), jnp.int32),
            pltpu.VMEM((N + L,), jnp.float32),
            pltpu.VMEM((N + L,), jnp.int32),
            pltpu.VMEM((NV,), jnp.int32),
            pltpu.VMEM((NV,), jnp.int32),
            pltpu.VMEM((NCHUNK,), jnp.float32),
            pltpu.VMEM((GH * K * 3,), jnp.float32),
        ],
    )(xt, ct)
    return out.reshape(B, G, K, 3)


def kernel(xyz):
    centers_gb3 = _fps_centers(xyz)
    center = jnp.transpose(centers_gb3, (1, 0, 2))
    patch = _knn_patch_sc(xyz, centers_gb3)
    return (patch, center)


# speedup vs baseline: 1.5949x; 1.5949x over previous
import functools

import jax
import jax.numpy as jnp
from jax import lax
from jax.experimental import pallas as pl
from jax.experimental.pallas import tpu as pltpu
from jax.experimental.pallas import tpu_sc as plsc

B, N, G, K = 16, 8192, 128, 32
L = 16
GH = G // 2
NV = N // L
CHUNK = 16
NCHUNK = NV // CHUNK
INF = 3.0e38


def _fps_body(xt_ref, centers_ref, dist_ref):
    X = xt_ref[0]
    Y = xt_ref[1]
    Z = xt_ref[2]
    lane = jax.lax.broadcasted_iota(jnp.int32, (B, N), 1)
    dist_ref[...] = jnp.full((B, N), 1e10, dtype=jnp.float32)

    def step(i, far):
        mask = lane == far
        cx = jnp.sum(jnp.where(mask, X, 0.0), axis=1, keepdims=True)
        cy = jnp.sum(jnp.where(mask, Y, 0.0), axis=1, keepdims=True)
        cz = jnp.sum(jnp.where(mask, Z, 0.0), axis=1, keepdims=True)
        centers_ref[i, :, :] = jnp.concatenate([cx, cy, cz], axis=1)
        dx = X - cx
        dy = Y - cy
        dz = Z - cz
        d = dx * dx + dy * dy + dz * dz
        nd = jnp.minimum(dist_ref[...], d)
        dist_ref[...] = nd
        m = jnp.max(nd, axis=1, keepdims=True)
        far2 = jnp.min(jnp.where(nd == m, lane, N), axis=1, keepdims=True)
        return far2

    jax.lax.fori_loop(0, G, step, jnp.zeros((B, 1), jnp.int32))


def _fps_centers(xyz):
    xt = jnp.transpose(xyz, (2, 0, 1))
    centers_gb3 = pl.pallas_call(
        _fps_body,
        out_shape=jax.ShapeDtypeStruct((G, B, 3), jnp.float32),
        scratch_shapes=[pltpu.VMEM((B, N), jnp.float32)],
    )(xt)
    return centers_gb3


def _knn_body(xt_hbm, ct_hbm, out_hbm, xv, yv, zv, cv, db, cdA, cixA, cdB, cixB, cdC, cixC, cdD, cixD, cd2, cix2, pbuf):
    c_ax = lax.axis_index("c")
    s_ax = lax.axis_index("s")
    b = s_ax
    gh = c_ax

    pltpu.sync_copy(xt_hbm.at[pl.ds(b * (3 * N), N)], xv)
    pltpu.sync_copy(xt_hbm.at[pl.ds(b * (3 * N) + N, N)], yv)
    pltpu.sync_copy(xt_hbm.at[pl.ds(b * (3 * N) + 2 * N, N)], zv)
    pltpu.sync_copy(ct_hbm.at[pl.ds(b * (3 * G), 3 * G)], cv)

    lane = lax.broadcasted_iota(jnp.int32, (L,), 0)
    inf_v = jnp.full((L,), INF, dtype=jnp.float32)
    bigi_v = jnp.full((L,), 1 << 30, dtype=jnp.int32)
    zero_i = jnp.zeros((L,), jnp.int32)

    def per_group(g, _):
        gg = gh * GH + g
        ggv = jnp.full((L,), 0, jnp.int32) + gg
        cgx = plsc.load_gather(cv, [ggv])
        cgy = plsc.load_gather(cv, [ggv + G])
        cgz = plsc.load_gather(cv, [ggv + 2 * G])

        def chunk1(c, carry):
            m1, m2 = carry
            base = c * (CHUNK * L)
            for v in range(CHUNK):
                sl = pl.ds(base + v * L, L)
                dx = xv[sl] - cgx
                dy = yv[sl] - cgy
                dz = zv[sl] - cgz
                d = dx * dx + dy * dy + dz * dz
                db[sl] = d
                m2 = jnp.minimum(m2, jnp.maximum(m1, d))
                m1 = jnp.minimum(m1, d)
            return m1, m2

        m1, m2 = lax.fori_loop(0, NCHUNK, chunk1, (inf_v, inf_v))
        tau = jnp.max(m2)

        bufs = ((cdA, cixA), (cdB, cixB), (cdC, cixC), (cdD, cixD))

        def chunk2(c, curs):
            curs = list(curs)
            base = c * (CHUNK * L)
            for v in range(CHUNK):
                w = v % 4
                sl = pl.ds(base + v * L, L)
                d = db[sl]
                msk = d <= tau
                bd, bi = bufs[w]
                plsc.store_compressed(bd.at[pl.ds(curs[w], L)], d, mask=msk)
                nvec = lane + (base + v * L)
                plsc.store_compressed(bi.at[pl.ds(curs[w], L)], nvec, mask=msk)
                curs[w] = curs[w] + plsc.all_reduce_population_count(msk)[0]
            return tuple(curs)

        curs = lax.fori_loop(0, NCHUNK, chunk2,
                             (jnp.int32(0),) * 4)

        cursor = jnp.int32(0)
        for w in range(4):
            bd, bi = bufs[w]
            bc = curs[w]

            def mg(v, cur2, bd=bd, bi=bi, bc=bc):
                d = bd[pl.ds(v * L, L)]
                iv = bi[pl.ds(v * L, L)]
                msk = (v * L + lane) < bc
                plsc.store_compressed(cd2.at[pl.ds(cur2, L)], d, mask=msk)
                plsc.store_compressed(cix2.at[pl.ds(cur2, L)], iv, mask=msk)
                return cur2 + plsc.all_reduce_population_count(msk)[0]

            cursor = lax.fori_loop(0, (bc + (L - 1)) // L, mg, cursor)

        plsc.store_scatter(cd2, [cursor + lane], inf_v)

        nvc = (cursor + (L - 1)) // L

        for t in range(K // L):
            accv = jnp.zeros((L,), jnp.int32)
            for jj in range(L):
                def scan(v, carry):
                    m, mc = carry
                    sl = pl.ds(v * L, L)
                    dv = cd2[sl]
                    cmb = cix2[sl] * 16384 + (lane + v * L)
                    upd = (dv < m) | ((dv == m) & (cmb < mc))
                    m = jnp.where(upd, dv, m)
                    mc = jnp.where(upd, cmb, mc)
                    return m, mc

                m, mc = lax.fori_loop(0, nvc, scan, (inf_v, bigi_v))
                dmin = jnp.min(m)
                cmbmin = jnp.min(jnp.where(m == dmin, mc, bigi_v))
                imin = cmbmin // 16384
                pmin = cmbmin - imin * 16384
                plsc.store_scatter(cd2, [zero_i + pmin], inf_v, mask=lane == 0)
                accv = jnp.where(lane == jj, imin, accv)

            px = plsc.load_gather(xv, [accv]) - cgx
            py = plsc.load_gather(yv, [accv]) - cgy
            pz = plsc.load_gather(zv, [accv]) - cgz
            pos = (g * K + t * L) * 3 + lane * 3
            plsc.store_scatter(pbuf, [pos], px)
            plsc.store_scatter(pbuf, [pos + 1], py)
            plsc.store_scatter(pbuf, [pos + 2], pz)
        return 0

    lax.fori_loop(0, GH, per_group, 0)
    pltpu.sync_copy(pbuf, out_hbm.at[pl.ds((b * 2 + gh) * (GH * K * 3), GH * K * 3)])


def _knn_patch_sc(xyz, centers_gb3):
    xt = jnp.transpose(xyz, (0, 2, 1)).reshape(B * 3 * N)
    ct = jnp.transpose(centers_gb3, (1, 2, 0)).reshape(B * 3 * G)
    mesh = plsc.VectorSubcoreMesh(core_axis_name="c", subcore_axis_name="s")
    out = pl.kernel(
        _knn_body,
        out_type=jax.ShapeDtypeStruct((B * 2 * GH * K * 3,), jnp.float32),
        mesh=mesh,
        compiler_params=pltpu.CompilerParams(needs_layout_passes=False),
        scratch_types=[
            pltpu.VMEM((N,), jnp.float32),
            pltpu.VMEM((N,), jnp.float32),
            pltpu.VMEM((N,), jnp.float32),
            pltpu.VMEM((3 * G,), jnp.float32),
            pltpu.VMEM((N,), jnp.float32),
            pltpu.VMEM((N // 4 + L,), jnp.float32),
            pltpu.VMEM((N // 4 + L,), jnp.int32),
            pltpu.VMEM((N // 4 + L,), jnp.float32),
            pltpu.VMEM((N // 4 + L,), jnp.int32),
            pltpu.VMEM((N // 4 + L,), jnp.float32),
            pltpu.VMEM((N // 4 + L,), jnp.int32),
            pltpu.VMEM((N // 4 + L,), jnp.float32),
            pltpu.VMEM((N // 4 + L,), jnp.int32),
            pltpu.VMEM((N + L,), jnp.float32),
            pltpu.VMEM((N + L,), jnp.int32),
            pltpu.VMEM((GH * K * 3,), jnp.float32),
        ],
    )(xt, ct)
    return out.reshape(B, G, K, 3)


def kernel(xyz):
    centers_gb3 = _fps_centers(xyz)
    center = jnp.transpose(centers_gb3, (1, 0, 2))
    patch = _knn_patch_sc(xyz, centers_gb3)
    return (patch, center)
